# Initial kernel scaffold; baseline (speedup 1.0000x reference)
import functools

import jax
import jax.numpy as jnp
from jax import lax
from jax.experimental import pallas as pl
from jax.experimental.pallas import tpu as pltpu
from jax.experimental.pallas import tpu_sc as plsc

N = 10000
E = 320000
D = 128

NC = 2
NS = 16
NW = NC * NS
EPW = E // NW
K = 80
NCHUNK = EPW // K
RPT = N // NS
CW = 16

_MESH = plsc.VectorSubcoreMesh(
    core_axis_name="c", subcore_axis_name="s", num_cores=NC, num_subcores=NS
)


def _sc_aggregate(with_count):
    out_type = [jax.ShapeDtypeStruct((NC, N, D), jnp.float32)]
    if with_count:
        out_type.append(jax.ShapeDtypeStruct((NC, N, CW), jnp.float32))
    scratch = [
        pltpu.VMEM((K,), jnp.int32),
        pltpu.VMEM((K,), jnp.int32),
        pltpu.VMEM((K, D), jnp.float32),
        pltpu.VMEM_SHARED((N, D), jnp.float32),
        pltpu.SemaphoreType.DMA,
    ]
    if with_count:
        scratch.insert(3, pltpu.VMEM((K, CW), jnp.float32))
        scratch.append(pltpu.VMEM_SHARED((N, CW), jnp.float32))

    @functools.partial(
        pl.kernel, out_type=out_type, mesh=_MESH, scratch_types=scratch
    )
    def body(*refs):
        if with_count:
            (feat_hbm, src_hbm, dst_hbm, zfeat_hbm, zcnt_hbm, ones_hbm,
             agg_out, cnt_out,
             src_v, dst_v, rows_v, ones_v, agg_sh, sem, cnt_sh) = refs
        else:
            (feat_hbm, src_hbm, dst_hbm, zfeat_hbm,
             agg_out,
             src_v, dst_v, rows_v, agg_sh, sem) = refs
        cid = lax.axis_index("c")
        sid = lax.axis_index("s")
        wid = cid * NS + sid
        rbase = sid * RPT

        pltpu.sync_copy(zfeat_hbm, agg_sh.at[pl.ds(rbase, RPT)])
        if with_count:
            pltpu.sync_copy(zcnt_hbm, cnt_sh.at[pl.ds(rbase, RPT)])
            pltpu.sync_copy(ones_hbm, ones_v)
        plsc.subcore_barrier()

        ebase = wid * EPW

        def chunk(i, carry):
            off = ebase + i * K
            pltpu.sync_copy(src_hbm.at[pl.ds(off, K)], src_v)
            pltpu.sync_copy(dst_hbm.at[pl.ds(off, K)], dst_v)
            pltpu.async_copy(feat_hbm.at[src_v], rows_v, sem).wait()
            pltpu.sync_copy(rows_v, agg_sh.at[dst_v], add=True)
            if with_count:
                pltpu.sync_copy(ones_v, cnt_sh.at[dst_v], add=True)
            return carry

        lax.fori_loop(0, NCHUNK, chunk, 0)
        plsc.subcore_barrier()

        pltpu.sync_copy(agg_sh.at[pl.ds(rbase, RPT)],
                        agg_out.at[cid, pl.ds(rbase, RPT)])
        if with_count:
            pltpu.sync_copy(cnt_sh.at[pl.ds(rbase, RPT)],
                            cnt_out.at[cid, pl.ds(rbase, RPT)])

    return body


_agg_with_cnt = _sc_aggregate(True)
_agg_no_cnt = _sc_aggregate(False)


def _combine(aggp, cntp, x, W_l, b, W_r, relu):
    BR = 2000
    grid = (N // BR,)

    def body(aggp_ref, cntp_ref, x_ref, wl_ref, b_ref, wr_ref, o_ref):
        agg = aggp_ref[0] + aggp_ref[1]
        cnt = cntp_ref[0, :, 0:1] + cntp_ref[1, :, 0:1]
        mean = agg / jnp.maximum(cnt, 1.0)
        acc = lax.dot_general(mean, wl_ref[...], (((1,), (1,)), ((), ())),
                              preferred_element_type=jnp.float32)
        acc = acc + lax.dot_general(x_ref[...], wr_ref[...],
                                    (((1,), (1,)), ((), ())),
                                    preferred_element_type=jnp.float32)
        acc = acc + b_ref[...]
        if relu:
            acc = jnp.maximum(acc, 0.0)
        o_ref[...] = acc

    return pl.pallas_call(
        body,
        grid=grid,
        in_specs=[
            pl.BlockSpec((NC, BR, D), lambda i: (0, i, 0)),
            pl.BlockSpec((NC, BR, CW), lambda i: (0, i, 0)),
            pl.BlockSpec((BR, D), lambda i: (i, 0)),
            pl.BlockSpec((D, D), lambda i: (0, 0)),
            pl.BlockSpec((1, D), lambda i: (0, 0)),
            pl.BlockSpec((D, D), lambda i: (0, 0)),
        ],
        out_specs=pl.BlockSpec((BR, D), lambda i: (i, 0)),
        out_shape=jax.ShapeDtypeStruct((N, D), jnp.float32),
    )(aggp, cntp, x, W_l, b.reshape(1, D), W_r)


def kernel(x, edge_index, W1_l, b1, W1_r, W2_l, b2, W2_r):
    src = edge_index[0].astype(jnp.int32)
    dst = edge_index[1].astype(jnp.int32)
    zfeat = jnp.zeros((RPT, D), jnp.float32)
    zcnt = jnp.zeros((RPT, CW), jnp.float32)
    ones_blk = jnp.zeros((K, CW), jnp.float32).at[:, 0].set(1.0)

    aggp1, cntp = _agg_with_cnt(x, src, dst, zfeat, zcnt, ones_blk)
    h = _combine(aggp1, cntp, x, W1_l, b1, W1_r, relu=True)
    (aggp2,) = _agg_no_cnt(h, src, dst, zfeat)
    out = _combine(aggp2, cntp, h, W2_l, b2, W2_r, relu=False)
    return out


# speedup vs baseline: 3.5561x; 3.5561x over previous
import functools

import jax
import jax.numpy as jnp
from jax import lax
from jax.experimental import pallas as pl
from jax.experimental.pallas import tpu as pltpu
from jax.experimental.pallas import tpu_sc as plsc

N = 10000
E = 320000
D = 128
HD = D // 2

NC = 2
NS = 16
EPT = E // NS
K = 80
NCHUNK = EPT // K
NP = 10240
RPT = NP // NS
CW = 16
L = 16

_MESH = plsc.VectorSubcoreMesh(
    core_axis_name="c", subcore_axis_name="s", num_cores=NC, num_subcores=NS
)


def _sc_aggregate(with_count):
    out_type = [jax.ShapeDtypeStruct((NC, NP, HD), jnp.float32)]
    if with_count:
        out_type.append(jax.ShapeDtypeStruct((NP, CW), jnp.float32))
    scratch = [
        pltpu.VMEM((K,), jnp.int32),
        pltpu.VMEM((K,), jnp.int32),
        pltpu.VMEM((K,), jnp.int32),
        pltpu.VMEM((K, HD), jnp.float32),
        pltpu.VMEM_SHARED((NP, HD), jnp.float32),
        pltpu.SemaphoreType.DMA,
    ]
    if with_count:
        scratch.insert(4, pltpu.VMEM((K, CW), jnp.float32))
        scratch.append(pltpu.VMEM_SHARED((NP, CW), jnp.float32))

    @functools.partial(
        pl.kernel, out_type=out_type, mesh=_MESH, scratch_types=scratch,
        compiler_params=pltpu.CompilerParams(use_tc_tiling_on_sc=False),
    )
    def body(*refs):
        if with_count:
            (feat_hbm, src_hbm, dst_hbm, zfeat_hbm, zcnt_hbm, ones_hbm,
             agg_out, cnt_out,
             src_v, idx_v, dst_v, rows_v, ones_v, agg_sh, sem, cnt_sh) = refs
        else:
            (feat_hbm, src_hbm, dst_hbm, zfeat_hbm,
             agg_out,
             src_v, idx_v, dst_v, rows_v, agg_sh, sem) = refs
        cid = lax.axis_index("c")
        sid = lax.axis_index("s")
        rbase = sid * RPT

        pltpu.sync_copy(zfeat_hbm, rows_v)
        for j in range(RPT // K):
            pltpu.sync_copy(rows_v, agg_sh.at[pl.ds(rbase + j * K, K)])
        if with_count:
            @pl.when(cid == 0)
            def _():
                pltpu.sync_copy(zcnt_hbm, ones_v)
                for j in range(RPT // K):
                    pltpu.sync_copy(ones_v, cnt_sh.at[pl.ds(rbase + j * K, K)])
                pltpu.sync_copy(ones_hbm, ones_v)
        plsc.subcore_barrier()

        ebase = sid * EPT

        def chunk(i, carry):
            off = ebase + i * K
            pltpu.sync_copy(src_hbm.at[pl.ds(off, K)], src_v)
            pltpu.sync_copy(dst_hbm.at[pl.ds(off, K)], dst_v)
            for j in range(K // L):
                sl = pl.ds(j * L, L)
                idx_v[sl] = src_v[sl] * 2 + cid
            pltpu.async_copy(feat_hbm.at[idx_v], rows_v, sem).wait()
            pltpu.sync_copy(rows_v, agg_sh.at[dst_v], add=True)
            if with_count:
                @pl.when(cid == 0)
                def _():
                    pltpu.sync_copy(ones_v, cnt_sh.at[dst_v], add=True)
            return carry

        lax.fori_loop(0, NCHUNK, chunk, 0)
        plsc.subcore_barrier()

        for j in range(RPT // K):
            pltpu.sync_copy(agg_sh.at[pl.ds(rbase + j * K, K)], rows_v)
            pltpu.sync_copy(rows_v, agg_out.at[cid, pl.ds(rbase + j * K, K)])
        if with_count:
            @pl.when(cid == 0)
            def _():
                for j in range(RPT // K):
                    pltpu.sync_copy(cnt_sh.at[pl.ds(rbase + j * K, K)], ones_v)
                    pltpu.sync_copy(ones_v, cnt_out.at[pl.ds(rbase + j * K, K)])

    return body


_agg_with_cnt = _sc_aggregate(True)
_agg_no_cnt = _sc_aggregate(False)


def _combine(aggp, cnt, x, W_l, b, W_r, relu):
    BR = 2000
    grid = (N // BR,)

    def body(aggp_ref, cnt_ref, x_ref, wl_ref, b_ref, wr_ref, o_ref):
        agg = jnp.concatenate([aggp_ref[0], aggp_ref[1]], axis=1)
        c = jnp.maximum(cnt_ref[:, 0:1], 1.0)
        mean = agg / c
        acc = lax.dot_general(mean, wl_ref[...], (((1,), (1,)), ((), ())),
                              preferred_element_type=jnp.float32)
        acc = acc + lax.dot_general(x_ref[...], wr_ref[...],
                                    (((1,), (1,)), ((), ())),
                                    preferred_element_type=jnp.float32)
        acc = acc + b_ref[...]
        if relu:
            acc = jnp.maximum(acc, 0.0)
        o_ref[...] = acc

    return pl.pallas_call(
        body,
        grid=grid,
        in_specs=[
            pl.BlockSpec((NC, BR, HD), lambda i: (0, i, 0)),
            pl.BlockSpec((BR, CW), lambda i: (i, 0)),
            pl.BlockSpec((BR, D), lambda i: (i, 0)),
            pl.BlockSpec((D, D), lambda i: (0, 0)),
            pl.BlockSpec((1, D), lambda i: (0, 0)),
            pl.BlockSpec((D, D), lambda i: (0, 0)),
        ],
        out_specs=pl.BlockSpec((BR, D), lambda i: (i, 0)),
        out_shape=jax.ShapeDtypeStruct((N, D), jnp.float32),
    )(aggp, cnt, x, W_l, b.reshape(1, D), W_r)


def kernel(x, edge_index, W1_l, b1, W1_r, W2_l, b2, W2_r):
    src = edge_index[0].astype(jnp.int32)
    dst = edge_index[1].astype(jnp.int32)
    zfeat = jnp.zeros((K, HD), jnp.float32)
    zcnt = jnp.zeros((K, CW), jnp.float32)
    ones_blk = jnp.zeros((K, CW), jnp.float32).at[:, 0].set(1.0)

    x2 = x.reshape(2 * N, HD)
    aggp1, cnt = _agg_with_cnt(x2, src, dst, zfeat, zcnt, ones_blk)
    h = _combine(aggp1, cnt, x, W1_l, b1, W1_r, relu=True)
    (aggp2,) = _agg_no_cnt(h.reshape(2 * N, HD), src, dst, zfeat)
    out = _combine(aggp2, cnt, h, W2_l, b2, W2_r, relu=False)
    return out


# speedup vs baseline: 9.2003x; 2.5872x over previous
import functools

import jax
import jax.numpy as jnp
from jax import lax
from jax.experimental import pallas as pl
from jax.experimental.pallas import tpu as pltpu
from jax.experimental.pallas import tpu_sc as plsc

N = 10000
E = 320000
D = 128
HD = D // 2

NC = 2
NS = 16
EPT = E // NS
K = 80
NCHUNK = EPT // K
NP = 10240
RPT = NP // NS
CW = 16
L = 16
BCH = 25
NBLK = NCHUNK // BCH

_MESH = plsc.VectorSubcoreMesh(
    core_axis_name="c", subcore_axis_name="s", num_cores=NC, num_subcores=NS
)


def _sc_aggregate(with_count):
    out_type = [jax.ShapeDtypeStruct((NC, NP, HD), jnp.float32)]
    if with_count:
        out_type.append(jax.ShapeDtypeStruct((NP, CW), jnp.float32))
    scratch = [
        pltpu.VMEM((BCH, K), jnp.int32),
        pltpu.VMEM((BCH, K), jnp.int32),
        pltpu.VMEM((2, K), jnp.int32),
        pltpu.VMEM((2, K, HD), jnp.float32),
        pltpu.VMEM_SHARED((NP, HD), jnp.float32),
        pltpu.SemaphoreType.DMA,
        pltpu.SemaphoreType.DMA,
        pltpu.SemaphoreType.DMA,
        pltpu.SemaphoreType.DMA,
    ]
    if with_count:
        scratch.insert(4, pltpu.VMEM((K, CW), jnp.float32))
        scratch.append(pltpu.VMEM_SHARED((NP, CW), jnp.float32))

    @functools.partial(
        pl.kernel, out_type=out_type, mesh=_MESH, scratch_types=scratch,
        compiler_params=pltpu.CompilerParams(use_tc_tiling_on_sc=False),
    )
    def body(*refs):
        if with_count:
            (feat_hbm, src_hbm, dst_hbm, zfeat_hbm, zcnt_hbm, ones_hbm,
             agg_out, cnt_out,
             src_blk, dst_blk, idx2, rows2, ones_v,
             agg_sh, sg0, sg1, ss0, ss1, cnt_sh) = refs
        else:
            (feat_hbm, src_hbm, dst_hbm, zfeat_hbm,
             agg_out,
             src_blk, dst_blk, idx2, rows2,
             agg_sh, sg0, sg1, ss0, ss1) = refs
        cid = lax.axis_index("c")
        sid = lax.axis_index("s")
        rbase = sid * RPT
        sem_g = (sg0, sg1)
        sem_s = (ss0, ss1)
        stage = rows2.at[0]

        pltpu.sync_copy(zfeat_hbm, stage)
        for j in range(RPT // K):
            pltpu.sync_copy(stage, agg_sh.at[pl.ds(rbase + j * K, K)])
        if with_count:
            pltpu.sync_copy(zcnt_hbm, ones_v)
            for j in range(RPT // K):
                pltpu.sync_copy(ones_v, cnt_sh.at[pl.ds(rbase + j * K, K)])
            pltpu.sync_copy(ones_hbm, ones_v)
        plsc.subcore_barrier()

        def scatters(q, j):
            ds_ = [pltpu.async_copy(rows2.at[q], agg_sh.at[dst_blk.at[j]],
                                    sem_s[q], add=True)]
            if with_count:
                ds_.append(pltpu.async_copy(ones_v, cnt_sh.at[dst_blk.at[j]],
                                            sem_s[q], add=True))
            return ds_

        def blk_body(b, carry):
            base = sid * NCHUNK + b * BCH
            pltpu.sync_copy(src_hbm.at[pl.ds(base, BCH)], src_blk)
            pltpu.sync_copy(dst_hbm.at[pl.ds(base, BCH)], dst_blk)
            g = [None, None]
            s = [None, None]
            for j in range(BCH):
                p = j & 1
                if s[p] is not None:
                    for d in s[p]:
                        d.wait()
                    s[p] = None
                for l in range(K // L):
                    sl = pl.ds(l * L, L)
                    idx2[p, sl] = src_blk[j, sl] * 2 + cid
                g[p] = pltpu.async_copy(feat_hbm.at[idx2.at[p]],
                                        rows2.at[p], sem_g[p])
                if j > 0:
                    q = 1 - p
                    g[q].wait()
                    g[q] = None
                    s[q] = scatters(q, j - 1)
            q = (BCH - 1) & 1
            if s[1 - q] is not None:
                for d in s[1 - q]:
                    d.wait()
            g[q].wait()
            for d in scatters(q, BCH - 1):
                d.wait()
            return carry

        lax.fori_loop(0, NBLK, blk_body, 0)
        plsc.subcore_barrier()

        for j in range(RPT // K):
            pltpu.sync_copy(agg_sh.at[pl.ds(rbase + j * K, K)], stage)
            pltpu.sync_copy(stage, agg_out.at[cid, pl.ds(rbase + j * K, K)])
        if with_count:
            @pl.when(cid == 0)
            def _():
                for j in range(RPT // K):
                    pltpu.sync_copy(cnt_sh.at[pl.ds(rbase + j * K, K)], ones_v)
                    pltpu.sync_copy(ones_v, cnt_out.at[pl.ds(rbase + j * K, K)])

    return body


_agg_with_cnt = _sc_aggregate(True)
_agg_no_cnt = _sc_aggregate(False)


def _combine(aggp, cnt, x, W_l, b, W_r, relu):
    BR = 2000
    grid = (N // BR,)

    def body(aggp_ref, cnt_ref, x_ref, wl_ref, b_ref, wr_ref, o_ref):
        agg = jnp.concatenate([aggp_ref[0], aggp_ref[1]], axis=1)
        c = jnp.maximum(cnt_ref[:, 0:1], 1.0)
        mean = agg / c
        acc = lax.dot_general(mean, wl_ref[...], (((1,), (1,)), ((), ())),
                              preferred_element_type=jnp.float32)
        acc = acc + lax.dot_general(x_ref[...], wr_ref[...],
                                    (((1,), (1,)), ((), ())),
                                    preferred_element_type=jnp.float32)
        acc = acc + b_ref[...]
        if relu:
            acc = jnp.maximum(acc, 0.0)
        o_ref[...] = acc

    return pl.pallas_call(
        body,
        grid=grid,
        in_specs=[
            pl.BlockSpec((NC, BR, HD), lambda i: (0, i, 0)),
            pl.BlockSpec((BR, CW), lambda i: (i, 0)),
            pl.BlockSpec((BR, D), lambda i: (i, 0)),
            pl.BlockSpec((D, D), lambda i: (0, 0)),
            pl.BlockSpec((1, D), lambda i: (0, 0)),
            pl.BlockSpec((D, D), lambda i: (0, 0)),
        ],
        out_specs=pl.BlockSpec((BR, D), lambda i: (i, 0)),
        out_shape=jax.ShapeDtypeStruct((N, D), jnp.float32),
    )(aggp, cnt, x, W_l, b.reshape(1, D), W_r)


def kernel(x, edge_index, W1_l, b1, W1_r, W2_l, b2, W2_r):
    src = edge_index[0].astype(jnp.int32)
    dst = edge_index[1].astype(jnp.int32)
    zfeat = jnp.zeros((K, HD), jnp.float32)
    zcnt = jnp.zeros((K, CW), jnp.float32)
    ones_blk = jnp.zeros((K, CW), jnp.float32).at[:, 0].set(1.0)

    src2 = src.reshape(E // K, K)
    dst2 = dst.reshape(E // K, K)
    x2 = x.reshape(2 * N, HD)
    aggp1, cnt = _agg_with_cnt(x2, src2, dst2, zfeat, zcnt, ones_blk)
    h = _combine(aggp1, cnt, x, W1_l, b1, W1_r, relu=True)
    (aggp2,) = _agg_no_cnt(h.reshape(2 * N, HD), src2, dst2, zfeat)
    out = _combine(aggp2, cnt, h, W2_l, b2, W2_r, relu=False)
    return out
